# baseline (speedup 1.0000x reference)
import functools

import jax
import jax.numpy as jnp
from jax import lax
from jax.experimental import pallas as pl
from jax.experimental.pallas import tpu as pltpu

B, N, W, R, C = 16, 1024, 64, 4, 2048
IF = R * W + R + W + 1 + W + W + R + 1 + 1 + 3 * R
BA = 2

_F32 = jnp.float32


def _sig(x):
    return 1.0 / (1.0 + jnp.exp(-x))


def _oneplus(x):
    return 1.0 + jnp.maximum(x, 0.0) + jnp.log(1.0 + jnp.exp(-jnp.abs(x)))


def _softmax_lanes(z):
    m = jnp.max(z, axis=-1, keepdims=True)
    e = jnp.exp(z - m)
    return e / jnp.sum(e, axis=-1, keepdims=True)


def _col(v):
    return jnp.transpose(v, (1, 0))


def _dot(a, b, dims):
    return lax.dot_general(a, b, (dims, ((), ())), preferred_element_type=_F32)


def _fused(x_ref, wif_ref, bif_ref, mem_ref, rw_ref, wwts_ref, us_ref,
           pr_ref, triu_ref, lm_ref,
           reads_out, memnew_out, rws_out, ww_out, usage_out, prec_out,
           link_out):
    b_id = pl.program_id(0)

    for k in range(BA):
        _one_batch(k, b_id, x_ref, wif_ref, bif_ref, mem_ref, rw_ref,
                   wwts_ref, us_ref, pr_ref, triu_ref, lm_ref, reads_out,
                   memnew_out, rws_out, ww_out, usage_out, prec_out, link_out)


def _one_batch(k, b_id, x_ref, wif_ref, bif_ref, mem_ref, rw_ref, wwts_ref,
               us_ref, pr_ref, triu_ref, lm_ref, reads_out, memnew_out,
               rws_out, ww_out, usage_out, prec_out, link_out):
    xrow = x_ref[pl.ds(b_id * BA + k, 1), :]
    iv = _dot(xrow, wif_ref[...], ((1,), (1,))) + bif_ref[...]

    def sl(a, b):
        return iv[:, a:b]

    mem = mem_ref[k]
    usage = us_ref[k]
    wwts = wwts_ref[k]

    ret = jnp.ones((1, N), _F32)
    for r in range(R):
        f = _sig(sl(453 + r, 454 + r))
        ret = ret * (1.0 - f * rw_ref[k, r:r + 1, :])
    u = (usage + wwts - usage * wwts) * ret
    usage_out[k] = u

    logu = jnp.log(jnp.maximum(u, 1e-30))
    ut = _col(u)
    maskf = (jnp.where(ut < u, 1.0, 0.0) +
             jnp.where(ut == u, triu_ref[...], 0.0))
    s = _dot(logu, maskf, ((1,), (0,)))
    alloc = (1.0 - u) * jnp.exp(s)

    wkey = sl(260, 324)
    dots = _dot(wkey, mem, ((1,), (1,)))
    onesw = jnp.ones((1, W), _F32)
    mn = jnp.sqrt(_dot(onesw, mem * mem, ((1,), (1,))))
    kn = jnp.sqrt(jnp.sum(wkey * wkey, axis=1, keepdims=True))
    cos = dots / (mn * kn + 1e-8)
    cw = _softmax_lanes(_oneplus(sl(324, 325)) * cos)

    ag = _sig(sl(457, 458))
    wg = _sig(sl(458, 459))
    ww = wg * (ag * alloc + (1.0 - ag) * cw)
    ww_out[k] = ww

    erase = _sig(sl(325, 389))
    wvec = sl(389, 453)
    wwt = _col(ww)
    memnew = mem * (1.0 - wwt * erase) + wwt * wvec
    memnew_out[k] = memnew

    prec = pr_ref[k]
    prec_out[k] = (1.0 - jnp.sum(ww, axis=1, keepdims=True)) * prec + ww

    link = (1.0 - wwt - ww) * lm_ref[k] + wwt * prec
    gi = lax.broadcasted_iota(jnp.int32, (N, N), 0)
    gj = lax.broadcasted_iota(jnp.int32, (N, N), 1)
    link = jnp.where(gi == gj, 0.0, link)
    link_out[k] = link

    prev = rw_ref[k]
    fw = _dot(prev, link, ((1,), (1,)))
    bw = _dot(prev, link, ((1,), (0,)))

    rk = jnp.concatenate([sl(64 * r, 64 * r + 64) for r in range(R)],
                         axis=0)
    dotsr = _dot(rk, memnew, ((1,), (1,)))
    mnn = jnp.sqrt(_dot(onesw, memnew * memnew, ((1,), (1,))))
    knr = jnp.sqrt(jnp.sum(rk * rk, axis=1, keepdims=True))
    cosr = dotsr / (mnn * knr + 1e-8)
    betar = _col(_oneplus(sl(256, 260)))
    c = _softmax_lanes(betar * cosr)

    mrow = jnp.concatenate([sl(459 + 3 * r, 462 + 3 * r) for r in range(R)],
                           axis=0)
    m = _softmax_lanes(mrow)
    rwv = m[:, 0:1] * bw + m[:, 1:2] * c + m[:, 2:3] * fw
    rws_out[k] = rwv
    reads_out[k] = _dot(rwv, memnew, ((1,), (0,)))


def kernel(x, memory, r_weights, w_weights, usage, precedence, link_matrix,
           W_if, b_if):
    f32 = jnp.float32
    bif2 = b_if.reshape(1, IF)
    ww3_in = w_weights.reshape(B, 1, N)
    us3 = usage.reshape(B, 1, N)
    pr3 = precedence.reshape(B, 1, N)

    triu = jnp.triu(jnp.ones((N, N), f32))

    (reads3, memory_n, rws, ww, usage_n, prec_n, link) = pl.pallas_call(
        _fused,
        grid=(B // BA,),
        in_specs=[
            pl.BlockSpec((B, C), lambda b: (0, 0)),
            pl.BlockSpec((IF, C), lambda b: (0, 0)),
            pl.BlockSpec((1, IF), lambda b: (0, 0)),
            pl.BlockSpec((BA, N, W), lambda b: (b, 0, 0)),
            pl.BlockSpec((BA, R, N), lambda b: (b, 0, 0)),
            pl.BlockSpec((BA, 1, N), lambda b: (b, 0, 0)),
            pl.BlockSpec((BA, 1, N), lambda b: (b, 0, 0)),
            pl.BlockSpec((BA, 1, N), lambda b: (b, 0, 0)),
            pl.BlockSpec((N, N), lambda b: (0, 0)),
            pl.BlockSpec((BA, N, N), lambda b: (b, 0, 0)),
        ],
        out_specs=[
            pl.BlockSpec((BA, R, W), lambda b: (b, 0, 0)),
            pl.BlockSpec((BA, N, W), lambda b: (b, 0, 0)),
            pl.BlockSpec((BA, R, N), lambda b: (b, 0, 0)),
            pl.BlockSpec((BA, 1, N), lambda b: (b, 0, 0)),
            pl.BlockSpec((BA, 1, N), lambda b: (b, 0, 0)),
            pl.BlockSpec((BA, 1, N), lambda b: (b, 0, 0)),
            pl.BlockSpec((BA, N, N), lambda b: (b, 0, 0)),
        ],
        out_shape=[
            jax.ShapeDtypeStruct((B, R, W), f32),
            jax.ShapeDtypeStruct((B, N, W), f32),
            jax.ShapeDtypeStruct((B, R, N), f32),
            jax.ShapeDtypeStruct((B, 1, N), f32),
            jax.ShapeDtypeStruct((B, 1, N), f32),
            jax.ShapeDtypeStruct((B, 1, N), f32),
            jax.ShapeDtypeStruct((B, N, N), f32),
        ],
        compiler_params=pltpu.CompilerParams(
            dimension_semantics=("parallel",)),
    )(x, W_if, bif2, memory, r_weights, ww3_in, us3, pr3, triu, link_matrix)

    reads = reads3.reshape(B, R * W)
    return (reads, memory_n, rws, ww.reshape(B, N), usage_n.reshape(B, N),
            prec_n.reshape(B, N), link)


# speedup vs baseline: 1.1272x; 1.1272x over previous
import functools

import jax
import jax.numpy as jnp
from jax import lax
from jax.experimental import pallas as pl
from jax.experimental.pallas import tpu as pltpu

B, N, W, R, C = 16, 1024, 64, 4, 2048
IF = R * W + R + W + 1 + W + W + R + 1 + 1 + 3 * R
BA = 2

_F32 = jnp.float32


def _sig(x):
    return 1.0 / (1.0 + jnp.exp(-x))


def _oneplus(x):
    return 1.0 + jnp.maximum(x, 0.0) + jnp.log(1.0 + jnp.exp(-jnp.abs(x)))


def _softmax_lanes(z):
    m = jnp.max(z, axis=-1, keepdims=True)
    e = jnp.exp(z - m)
    return e / jnp.sum(e, axis=-1, keepdims=True)


def _col(v):
    return jnp.transpose(v, (1, 0))


def _dot(a, b, dims):
    return lax.dot_general(a, b, (dims, ((), ())), preferred_element_type=_F32)


def _fused(x_ref, wif_ref, bif_ref, mem_ref, rw_ref, wwts_ref, us_ref,
           pr_ref, triu_ref, lm_ref,
           reads_out, memnew_out, rws_out, ww_out, usage_out, prec_out,
           link_out, iv_scr):
    b_id = pl.program_id(0)

    @pl.when(b_id == 0)
    def _():
        iv_scr[...] = _dot(x_ref[...], wif_ref[...], ((1,), (1,))) + bif_ref[...]

    for k in range(BA):
        _one_batch(k, b_id, mem_ref, rw_ref, wwts_ref, us_ref, pr_ref,
                   triu_ref, lm_ref, reads_out, memnew_out, rws_out, ww_out,
                   usage_out, prec_out, link_out, iv_scr)


def _one_batch(k, b_id, mem_ref, rw_ref, wwts_ref, us_ref, pr_ref, triu_ref,
               lm_ref, reads_out, memnew_out, rws_out, ww_out, usage_out,
               prec_out, link_out, iv_scr):
    iv = iv_scr[pl.ds(b_id * BA + k, 1), :]

    def sl(a, b):
        return iv[:, a:b]

    mem = mem_ref[k]
    usage = us_ref[k]
    wwts = wwts_ref[k]

    ret = jnp.ones((1, N), _F32)
    for r in range(R):
        f = _sig(sl(453 + r, 454 + r))
        ret = ret * (1.0 - f * rw_ref[k, r:r + 1, :])
    u = (usage + wwts - usage * wwts) * ret
    usage_out[k] = u

    logu = jnp.log(jnp.maximum(u, 1e-30))
    ut = _col(u)
    maskf = (jnp.where(ut < u, 1.0, 0.0) +
             jnp.where(ut == u, triu_ref[...], 0.0))
    s = _dot(logu, maskf, ((1,), (0,)))
    alloc = (1.0 - u) * jnp.exp(s)

    wkey = sl(260, 324)
    dots = _dot(wkey, mem, ((1,), (1,)))
    onesw = jnp.ones((1, W), _F32)
    mn = jnp.sqrt(_dot(onesw, mem * mem, ((1,), (1,))))
    kn = jnp.sqrt(jnp.sum(wkey * wkey, axis=1, keepdims=True))
    cos = dots / (mn * kn + 1e-8)
    cw = _softmax_lanes(_oneplus(sl(324, 325)) * cos)

    ag = _sig(sl(457, 458))
    wg = _sig(sl(458, 459))
    ww = wg * (ag * alloc + (1.0 - ag) * cw)
    ww_out[k] = ww

    erase = _sig(sl(325, 389))
    wvec = sl(389, 453)
    wwt = _col(ww)
    memnew = mem * (1.0 - wwt * erase) + wwt * wvec
    memnew_out[k] = memnew

    prec = pr_ref[k]
    prec_out[k] = (1.0 - jnp.sum(ww, axis=1, keepdims=True)) * prec + ww

    link = (1.0 - wwt - ww) * lm_ref[k] + wwt * prec
    gi = lax.broadcasted_iota(jnp.int32, (N, N), 0)
    gj = lax.broadcasted_iota(jnp.int32, (N, N), 1)
    link = jnp.where(gi == gj, 0.0, link)
    link_out[k] = link

    prev = rw_ref[k]
    fw = _dot(prev, link, ((1,), (1,)))
    bw = _dot(prev, link, ((1,), (0,)))

    rk = jnp.concatenate([sl(64 * r, 64 * r + 64) for r in range(R)],
                         axis=0)
    dotsr = _dot(rk, memnew, ((1,), (1,)))
    mnn = jnp.sqrt(_dot(onesw, memnew * memnew, ((1,), (1,))))
    knr = jnp.sqrt(jnp.sum(rk * rk, axis=1, keepdims=True))
    cosr = dotsr / (mnn * knr + 1e-8)
    betar = _col(_oneplus(sl(256, 260)))
    c = _softmax_lanes(betar * cosr)

    mrow = jnp.concatenate([sl(459 + 3 * r, 462 + 3 * r) for r in range(R)],
                           axis=0)
    m = _softmax_lanes(mrow)
    rwv = m[:, 0:1] * bw + m[:, 1:2] * c + m[:, 2:3] * fw
    rws_out[k] = rwv
    reads_out[k] = _dot(rwv, memnew, ((1,), (0,)))


def kernel(x, memory, r_weights, w_weights, usage, precedence, link_matrix,
           W_if, b_if):
    f32 = jnp.float32
    bif2 = b_if.reshape(1, IF)
    ww3_in = w_weights.reshape(B, 1, N)
    us3 = usage.reshape(B, 1, N)
    pr3 = precedence.reshape(B, 1, N)

    triu = jnp.triu(jnp.ones((N, N), f32))

    (reads3, memory_n, rws, ww, usage_n, prec_n, link) = pl.pallas_call(
        _fused,
        grid=(B // BA,),
        in_specs=[
            pl.BlockSpec((B, C), lambda b: (0, 0)),
            pl.BlockSpec((IF, C), lambda b: (0, 0)),
            pl.BlockSpec((1, IF), lambda b: (0, 0)),
            pl.BlockSpec((BA, N, W), lambda b: (b, 0, 0)),
            pl.BlockSpec((BA, R, N), lambda b: (b, 0, 0)),
            pl.BlockSpec((BA, 1, N), lambda b: (b, 0, 0)),
            pl.BlockSpec((BA, 1, N), lambda b: (b, 0, 0)),
            pl.BlockSpec((BA, 1, N), lambda b: (b, 0, 0)),
            pl.BlockSpec((N, N), lambda b: (0, 0)),
            pl.BlockSpec((BA, N, N), lambda b: (b, 0, 0)),
        ],
        out_specs=[
            pl.BlockSpec((BA, R, W), lambda b: (b, 0, 0)),
            pl.BlockSpec((BA, N, W), lambda b: (b, 0, 0)),
            pl.BlockSpec((BA, R, N), lambda b: (b, 0, 0)),
            pl.BlockSpec((BA, 1, N), lambda b: (b, 0, 0)),
            pl.BlockSpec((BA, 1, N), lambda b: (b, 0, 0)),
            pl.BlockSpec((BA, 1, N), lambda b: (b, 0, 0)),
            pl.BlockSpec((BA, N, N), lambda b: (b, 0, 0)),
        ],
        out_shape=[
            jax.ShapeDtypeStruct((B, R, W), f32),
            jax.ShapeDtypeStruct((B, N, W), f32),
            jax.ShapeDtypeStruct((B, R, N), f32),
            jax.ShapeDtypeStruct((B, 1, N), f32),
            jax.ShapeDtypeStruct((B, 1, N), f32),
            jax.ShapeDtypeStruct((B, 1, N), f32),
            jax.ShapeDtypeStruct((B, N, N), f32),
        ],
        scratch_shapes=[pltpu.VMEM((B, IF), f32)],
        compiler_params=pltpu.CompilerParams(
            dimension_semantics=("arbitrary",)),
    )(x, W_if, bif2, memory, r_weights, ww3_in, us3, pr3, triu, link_matrix)

    reads = reads3.reshape(B, R * W)
    return (reads, memory_n, rws, ww.reshape(B, N), usage_n.reshape(B, N),
            prec_n.reshape(B, N), link)
